# baseline (speedup 1.0000x reference)
import functools

import jax
import jax.numpy as jnp
from jax import lax
from jax.experimental import pallas as pl
from jax.experimental.pallas import tpu as pltpu
from jax.experimental.pallas import tpu_sc as plsc

BS, NV, SEQ = 4096, 26, 200
NCAT = 10
VOCAB = 1000
LANES = 16
NC, NS = 2, 16
NWORKERS = NC * NS
BW = BS // NWORKERS
NBUF = 3


def _sc_body(x_hbm, tab_hbm, out_hbm, tab_v, bufs, sin, sout, spass):
    wid = lax.axis_index("s") * NC + lax.axis_index("c")
    b0 = wid * BW

    pass_copy = pltpu.make_async_copy(
        x_hbm.at[pl.ds(NCAT, NV - NCAT), :, pl.ds(b0, BW)],
        out_hbm.at[pl.ds(NCAT, NV - NCAT), :, pl.ds(b0, BW)],
        spass)
    pass_copy.start()

    pltpu.sync_copy(tab_hbm, tab_v)

    def in_copy(c, b):
        return pltpu.make_async_copy(
            x_hbm.at[pl.ds(c, 1), :, pl.ds(b0, BW)], bufs[b], sin[b])

    def out_copy(c, b):
        return pltpu.make_async_copy(
            bufs[b], out_hbm.at[pl.ds(c, 1), :, pl.ds(b0, BW)], sout[b])

    def gather_plane(c, b):
        buf = bufs[b]
        off = jnp.int32(c * VOCAB)

        def s_body(s, carry):
            for k in range(BW // LANES):
                sl = (0, s, pl.ds(k * LANES, LANES))
                idx = buf[sl].astype(jnp.int32) + off
                buf[sl] = plsc.load_gather(tab_v, [idx])
            return carry

        lax.fori_loop(0, SEQ, s_body, 0, unroll=False)

    for c in range(NBUF - 1):
        in_copy(c, c).start()
    for c in range(NCAT):
        b = c % NBUF
        nxt = c + NBUF - 1
        if nxt < NCAT:
            bn = nxt % NBUF
            if c >= 1:
                out_copy(c - 1, bn).wait()
            in_copy(nxt, bn).start()
        in_copy(c, b).wait()
        gather_plane(c, b)
        out_copy(c, b).start()
    for c in range(NCAT - NBUF, NCAT):
        out_copy(c, c % NBUF).wait()
    pass_copy.wait()


@functools.partial(jax.jit, static_argnames=())
def _run(xt, tab):
    mesh = plsc.VectorSubcoreMesh(core_axis_name="c", subcore_axis_name="s")
    return pl.kernel(
        lambda x_, t, o, tv, b0, b1, b2, si0, si1, si2, so0, so1, so2, sp:
            _sc_body(x_, t, o, tv, (b0, b1, b2),
                     (si0, si1, si2), (so0, so1, so2), sp),
        out_type=jax.ShapeDtypeStruct((NV, SEQ, BS), jnp.float32),
        mesh=mesh,
        scratch_types=[pltpu.VMEM((NCAT * VOCAB,), jnp.float32)]
        + [pltpu.VMEM((1, SEQ, BW), jnp.float32)] * NBUF
        + [pltpu.SemaphoreType.DMA] * (2 * NBUF + 1),
        compiler_params=pltpu.CompilerParams(
            needs_layout_passes=False, use_tc_tiling_on_sc=True),
    )(xt, tab)


def kernel(x, W0, W1, W2, W3, W4, W5, W6, W7, W8, W9):
    tab = jnp.concatenate(
        [W0, W1, W2, W3, W4, W5, W6, W7, W8, W9], axis=0
    ).reshape(NCAT * VOCAB)
    out_t = _run(jnp.transpose(x, (1, 2, 0)), tab)
    return jnp.transpose(out_t, (2, 0, 1))


# speedup vs baseline: 9.2442x; 9.2442x over previous
import functools

import jax
import jax.numpy as jnp
from jax import lax
from jax.experimental import pallas as pl
from jax.experimental.pallas import tpu as pltpu
from jax.experimental.pallas import tpu_sc as plsc

BS, NV, SEQ = 4096, 26, 200
NCAT = 10
VOCAB = 1000
LANES = 16
NC, NS = 2, 16
NWORKERS = NC * NS
NSEG = NV * (SEQ // 8)
GSEG = NCAT * (SEQ // 8)
SEGW = 8 * BS
NBUF = 3
NSTEP = (NSEG + NWORKERS - 1) // NWORKERS


def _sc_body(x_hbm, tab_hbm, out_hbm, tab_v, bufs, sin, sout):
    wid = lax.axis_index("s") * NC + lax.axis_index("c")
    pltpu.sync_copy(tab_hbm, tab_v)

    def seg(i):
        return wid + i * NWORKERS

    def in_copy(i, b):
        return pltpu.make_async_copy(
            x_hbm.at[pl.ds(seg(i), 1)], bufs[b], sin[b])

    def out_copy(i, b):
        return pltpu.make_async_copy(
            bufs[b], out_hbm.at[pl.ds(seg(i), 1)], sout[b])

    def gather_seg(i, b):
        buf = bufs[b]
        m = seg(i)
        off = ((m * 41944) >> 20) * VOCAB

        def k_body(k, carry):
            for s in range(8):
                sl = (0, s, pl.ds(k * LANES, LANES))
                idx = buf[sl].astype(jnp.int32) + off
                buf[sl] = plsc.load_gather(tab_v, [idx])
            return carry

        lax.fori_loop(0, BS // LANES, k_body, 0, unroll=False)

    def step(i, b):
        bp = (b - 1) % NBUF
        if i >= 1:
            @pl.when(seg(i - 1) < NSEG)
            def _():
                out_copy(i - 1, bp).wait()

        @pl.when(seg(i + NBUF - 1) < NSEG)
        def _():
            in_copy(i + NBUF - 1, bp).start()

        @pl.when(seg(i) < NSEG)
        def _():
            in_copy(i, b).wait()

            @pl.when(seg(i) < GSEG)
            def _():
                gather_seg(i, b)
            out_copy(i, b).start()

    for i in range(NBUF - 1):
        @pl.when(seg(i) < NSEG)
        def _():
            in_copy(i, i).start()

    for i in range(NSTEP):
        step(i, i % NBUF)

    @pl.when(seg(NSTEP - 1) < NSEG)
    def _():
        out_copy(NSTEP - 1, (NSTEP - 1) % NBUF).wait()


@functools.partial(jax.jit, static_argnames=())
def _run(xs, tab):
    mesh = plsc.VectorSubcoreMesh(core_axis_name="c", subcore_axis_name="s")
    return pl.kernel(
        lambda x_, t, o, tv, b0, b1, b2, si0, si1, si2, so0, so1, so2:
            _sc_body(x_, t, o, tv, (b0, b1, b2),
                     (si0, si1, si2), (so0, so1, so2)),
        out_type=jax.ShapeDtypeStruct((NSEG, 8, BS), jnp.float32),
        mesh=mesh,
        scratch_types=[pltpu.VMEM((NCAT * VOCAB,), jnp.float32)]
        + [pltpu.VMEM((1, 8, BS), jnp.float32)] * NBUF
        + [pltpu.SemaphoreType.DMA] * (2 * NBUF),
        compiler_params=pltpu.CompilerParams(
            needs_layout_passes=False, use_tc_tiling_on_sc=True),
    )(xs, tab)


def kernel(x, W0, W1, W2, W3, W4, W5, W6, W7, W8, W9):
    tab = jnp.concatenate(
        [W0, W1, W2, W3, W4, W5, W6, W7, W8, W9], axis=0
    ).reshape(NCAT * VOCAB)
    xs = jnp.transpose(x, (1, 2, 0)).reshape(NSEG, 8, BS)
    out_s = _run(xs, tab)
    return jnp.transpose(out_s.reshape(NV, SEQ, BS), (2, 0, 1))


# speedup vs baseline: 9.5597x; 1.0341x over previous
import functools

import jax
import jax.numpy as jnp
from jax import lax
from jax.experimental import pallas as pl
from jax.experimental.pallas import tpu as pltpu
from jax.experimental.pallas import tpu_sc as plsc

BS, NV, SEQ = 4096, 26, 200
NCAT = 10
VOCAB = 1000
LANES = 16
NC, NS = 2, 16
NWORKERS = NC * NS
NSEG = NV * (SEQ // 8)
GSEG = NCAT * (SEQ // 8)
SEGW = 8 * BS
NBUF = 3
NSTEP = (NSEG + NWORKERS - 1) // NWORKERS


def _sc_body(x_hbm, tab_hbm, out_hbm, tab_v, bufs, sin, sout):
    wid = lax.axis_index("s") * NC + lax.axis_index("c")
    pltpu.sync_copy(tab_hbm, tab_v)

    def seg(i):
        return wid + i * NWORKERS

    def in_copy(i, b):
        return pltpu.make_async_copy(
            x_hbm.at[pl.ds(seg(i), 1)], bufs[b], sin[b])

    def out_copy(i, b):
        return pltpu.make_async_copy(
            bufs[b], out_hbm.at[pl.ds(seg(i), 1)], sout[b])

    def gather_seg(i, b):
        buf = bufs[b]
        m = seg(i)
        off = ((m * 41944) >> 20) * VOCAB

        def k_body(k, carry):
            for s in range(8):
                sl = (0, s, pl.ds(k * LANES, LANES))
                idx = buf[sl].astype(jnp.int32) + off
                buf[sl] = plsc.load_gather(tab_v, [idx])
            return carry

        lax.fori_loop(0, BS // LANES, k_body, 0, unroll=False)

    def step(i, b):
        @pl.when(seg(i) < NSEG)
        def _():
            in_copy(i, b).wait()

            @pl.when(seg(i) < GSEG)
            def _():
                gather_seg(i, b)
            out_copy(i, b).start()

        bp = (b - 1) % NBUF
        if i >= 1:
            @pl.when(seg(i - 1) < NSEG)
            def _():
                out_copy(i - 1, bp).wait()

        @pl.when(seg(i + NBUF - 1) < NSEG)
        def _():
            in_copy(i + NBUF - 1, bp).start()

    for i in range(NBUF - 1):
        @pl.when(seg(i) < NSEG)
        def _():
            in_copy(i, i).start()

    for i in range(NSTEP):
        step(i, i % NBUF)

    @pl.when(seg(NSTEP - 1) < NSEG)
    def _():
        out_copy(NSTEP - 1, (NSTEP - 1) % NBUF).wait()


@functools.partial(jax.jit, static_argnames=())
def _run(xs, tab):
    mesh = plsc.VectorSubcoreMesh(core_axis_name="c", subcore_axis_name="s")
    return pl.kernel(
        lambda x_, t, o, tv, b0, b1, b2, si0, si1, si2, so0, so1, so2:
            _sc_body(x_, t, o, tv, (b0, b1, b2),
                     (si0, si1, si2), (so0, so1, so2)),
        out_type=jax.ShapeDtypeStruct((NSEG, 8, BS), jnp.float32),
        mesh=mesh,
        scratch_types=[pltpu.VMEM((NCAT * VOCAB,), jnp.float32)]
        + [pltpu.VMEM((1, 8, BS), jnp.float32)] * NBUF
        + [pltpu.SemaphoreType.DMA] * (2 * NBUF),
        compiler_params=pltpu.CompilerParams(
            needs_layout_passes=False, use_tc_tiling_on_sc=True),
    )(xs, tab)


def kernel(x, W0, W1, W2, W3, W4, W5, W6, W7, W8, W9):
    tab = jnp.concatenate(
        [W0, W1, W2, W3, W4, W5, W6, W7, W8, W9], axis=0
    ).reshape(NCAT * VOCAB)
    xs = jnp.transpose(x, (1, 2, 0)).reshape(NSEG, 8, BS)
    out_s = _run(xs, tab)
    return jnp.transpose(out_s.reshape(NV, SEQ, BS), (2, 0, 1))
